# Initial kernel scaffold; baseline (speedup 1.0000x reference)
import jax
import jax.numpy as jnp
from jax.experimental import pallas as pl


def kernel(segments, table):
    raise NotImplementedError("write your pallas kernel here")


# speedup vs baseline: 3.7310x; 3.7310x over previous
import jax
import jax.numpy as jnp
from jax.experimental import pallas as pl

_T = 1024


def _body(seg_ref, table_ref, out_ref):
    seg = seg_ref[...]
    row0 = table_ref[0:1, :]
    row1 = table_ref[1:2, :]
    out_ref[...] = jnp.where(seg == 0, row0, row1)


def kernel(segments, table):
    B, S = segments.shape
    H = table.shape[1]
    N = B * S
    seg2 = segments.reshape(N, 1)
    out = pl.pallas_call(
        _body,
        grid=(N // _T,),
        in_specs=[
            pl.BlockSpec((_T, 1), lambda i: (i, 0)),
            pl.BlockSpec((2, H), lambda i: (0, 0)),
        ],
        out_specs=pl.BlockSpec((_T, H), lambda i: (i, 0)),
        out_shape=jax.ShapeDtypeStruct((N, H), table.dtype),
    )(seg2, table)
    return out.reshape(B, S, H)


# speedup vs baseline: 3.9716x; 1.0645x over previous
import jax
import jax.numpy as jnp
from jax.experimental import pallas as pl
from jax.experimental.pallas import tpu as pltpu

_T = 2048


def _body(seg_ref, table_ref, out_ref):
    seg = seg_ref[...]
    row0 = table_ref[0:1, :]
    row1 = table_ref[1:2, :]
    out_ref[...] = jnp.where(seg == 0, row0, row1)


def kernel(segments, table):
    B, S = segments.shape
    H = table.shape[1]
    N = B * S
    seg2 = segments.reshape(N, 1)
    out = pl.pallas_call(
        _body,
        grid=(N // _T,),
        in_specs=[
            pl.BlockSpec((_T, 1), lambda i: (i, 0)),
            pl.BlockSpec((2, H), lambda i: (0, 0)),
        ],
        out_specs=pl.BlockSpec((_T, H), lambda i: (i, 0)),
        out_shape=jax.ShapeDtypeStruct((N, H), table.dtype),
        compiler_params=pltpu.CompilerParams(
            dimension_semantics=("parallel",),
        ),
    )(seg2, table)
    return out.reshape(B, S, H)


# speedup vs baseline: 4.0290x; 1.0145x over previous
import jax
import jax.numpy as jnp
from jax.experimental import pallas as pl
from jax.experimental.pallas import tpu as pltpu

_T = 4096


def _body(seg_ref, table_ref, out_ref):
    seg = seg_ref[...]
    row0 = table_ref[0:1, :]
    row1 = table_ref[1:2, :]
    out_ref[...] = jnp.where(seg == 0, row0, row1)


def kernel(segments, table):
    B, S = segments.shape
    H = table.shape[1]
    N = B * S
    seg2 = segments.reshape(N, 1)
    out = pl.pallas_call(
        _body,
        grid=(N // _T,),
        in_specs=[
            pl.BlockSpec((_T, 1), lambda i: (i, 0)),
            pl.BlockSpec((2, H), lambda i: (0, 0)),
        ],
        out_specs=pl.BlockSpec((_T, H), lambda i: (i, 0)),
        out_shape=jax.ShapeDtypeStruct((N, H), table.dtype),
        compiler_params=pltpu.CompilerParams(
            dimension_semantics=("parallel",),
        ),
    )(seg2, table)
    return out.reshape(B, S, H)
